# baseline (speedup 1.0000x reference)
import jax
import jax.numpy as jnp
from jax.experimental import pallas as pl
from jax.experimental.pallas import tpu as pltpu

_N = 20
_P = _N // 2
_F_IN = 16
_F_OUT = 32
_EMB = 4
_FC = _F_IN + _F_OUT


def _agcrn_body(x_ref, h_ref, e_ref, gw_ref, gb_ref, uw_ref, ub_ref,
                lwb_ref, lb_ref, out_ref,
                gwx_s, gwh_s, gb_s, uwx_s, uwh_s, ub_s):
    f, g = _F_IN, _F_OUT

    def _expand_weights():
        gwx_s[...] = jnp.zeros_like(gwx_s)
        gwh_s[...] = jnp.zeros_like(gwh_s)
        uwx_s[...] = jnp.zeros_like(uwx_s)
        uwh_s[...] = jnp.zeros_like(uwh_s)

        def expand(pool_ref, n):
            w = e_ref[n, 0] * pool_ref[0]
            for d in range(1, _EMB):
                w = w + e_ref[n, d] * pool_ref[d]
            return w

        for p in range(_P):
            n0, n1 = 2 * p, 2 * p + 1
            wg0 = 0.5 * expand(gw_ref, n0)
            wg1 = 0.5 * expand(gw_ref, n1)
            bg0 = 0.5 * expand(gb_ref, n0)
            bg1 = 0.5 * expand(gb_ref, n1)
            wu0 = expand(uw_ref, n0)
            wu1 = expand(uw_ref, n1)
            bu0 = expand(ub_ref, n0)
            bu1 = expand(ub_ref, n1)

            gwx_s[p, 0 * g:1 * g, 0:f] = wg0[0:g, 0:f]
            gwx_s[p, 1 * g:2 * g, f:2 * f] = wg1[0:g, 0:f]
            gwx_s[p, 2 * g:3 * g, 0:f] = wg0[g:2 * g, 0:f]
            gwx_s[p, 3 * g:4 * g, f:2 * f] = wg1[g:2 * g, 0:f]
            gwh_s[p, 0 * g:1 * g, 0:g] = wg0[0:g, f:]
            gwh_s[p, 1 * g:2 * g, g:2 * g] = wg1[0:g, f:]
            gwh_s[p, 2 * g:3 * g, 0:g] = wg0[g:2 * g, f:]
            gwh_s[p, 3 * g:4 * g, g:2 * g] = wg1[g:2 * g, f:]
            gb_s[p, 0 * g:1 * g] = bg0[0:g]
            gb_s[p, 1 * g:2 * g] = bg1[0:g]
            gb_s[p, 2 * g:3 * g] = bg0[g:2 * g]
            gb_s[p, 3 * g:4 * g] = bg1[g:2 * g]

            uwx_s[p, 0:g, 0:f] = wu0[:, 0:f]
            uwx_s[p, g:2 * g, f:2 * f] = wu1[:, 0:f]
            uwh_s[p, 0:g, 0:g] = 0.5 * wu0[:, f:]
            uwh_s[p, g:2 * g, g:2 * g] = 0.5 * wu1[:, f:]
            ub_s[p, 0:g] = bu0
            ub_s[p, g:2 * g] = bu1

    _expand_weights()

    xt = x_ref[...]
    ht = h_ref[...]

    vs = []
    for p in range(_P):
        xp = xt[2 * f * p:2 * f * (p + 1)]
        hp = ht[2 * g * p:2 * g * (p + 1)]

        ga = (jnp.dot(gwx_s[p], xp, preferred_element_type=jnp.float32)
              + jnp.dot(gwh_s[p], hp, preferred_element_type=jnp.float32)
              + gb_s[p])
        t = jnp.tanh(ga)
        tz = t[:2 * g]
        tr = t[2 * g:]
        u = tz * hp + hp
        hc = jnp.tanh(
            jnp.dot(uwx_s[p], xp, preferred_element_type=jnp.float32)
            + jnp.dot(uwh_s[p], u, preferred_element_type=jnp.float32)
            + ub_s[p])
        vs.append(jax.nn.relu((hc + hp) + tr * (hp - hc)))

    hall = jnp.concatenate(vs, axis=0)
    out_ref[...] = (
        jnp.dot(lwb_ref[...], hall, preferred_element_type=jnp.float32)
        + lb_ref[0, 0])


def kernel(x, e, h, gate_W, gate_b, update_W, update_b, lin_W, lin_b):
    B = x.shape[0]
    BB = 4096
    x2 = x.transpose(1, 2, 0).reshape(_N * _F_IN, B)
    h2 = h.transpose(1, 2, 0).reshape(_N * _F_OUT, B)
    gwT = gate_W.reshape(_EMB, _FC, 2 * _F_OUT).transpose(0, 2, 1)
    uwT = update_W.reshape(_EMB, _FC, _F_OUT).transpose(0, 2, 1)
    gbT = gate_b.reshape(_EMB, 2 * _F_OUT, 1)
    ubT = update_b.reshape(_EMB, _F_OUT, 1)
    lw_big = 0.5 * jnp.kron(jnp.eye(_N, dtype=x.dtype), lin_W)
    lb = lin_b.reshape(1, 1)

    grid = (B // BB,)
    full = lambda shape: pl.BlockSpec(shape, lambda i: (0,) * len(shape))
    out_t = pl.pallas_call(
        _agcrn_body,
        grid=grid,
        in_specs=[
            pl.BlockSpec((_N * _F_IN, BB), lambda i: (0, i)),
            pl.BlockSpec((_N * _F_OUT, BB), lambda i: (0, i)),
            full(e.shape),
            full(gwT.shape),
            full(gbT.shape),
            full(uwT.shape),
            full(ubT.shape),
            full(lw_big.shape),
            full(lb.shape),
        ],
        out_specs=pl.BlockSpec((_N, BB), lambda i: (0, i)),
        out_shape=jax.ShapeDtypeStruct((_N, B), x.dtype),
        scratch_shapes=[
            pltpu.VMEM((_P, 4 * _F_OUT, 2 * _F_IN), jnp.float32),
            pltpu.VMEM((_P, 4 * _F_OUT, 2 * _F_OUT), jnp.float32),
            pltpu.VMEM((_P, 4 * _F_OUT, 1), jnp.float32),
            pltpu.VMEM((_P, 2 * _F_OUT, 2 * _F_IN), jnp.float32),
            pltpu.VMEM((_P, 2 * _F_OUT, 2 * _F_OUT), jnp.float32),
            pltpu.VMEM((_P, 2 * _F_OUT, 1), jnp.float32),
        ],
        compiler_params=pltpu.CompilerParams(
            dimension_semantics=("parallel",)),
    )(x2, h2, e, gwT, gbT, uwT, ubT, lw_big, lb)
    return out_t.T.reshape(B, _N, 1)


# speedup vs baseline: 1.0101x; 1.0101x over previous
import jax
import jax.numpy as jnp
from jax.experimental import pallas as pl
from jax.experimental.pallas import tpu as pltpu

_N = 20
_P = _N // 2
_F_IN = 16
_F_OUT = 32
_EMB = 4
_FC = _F_IN + _F_OUT


def _agcrn_body(x_ref, h_ref, e_ref, gw_ref, gb_ref, uw_ref, ub_ref,
                lwb_ref, lb_ref, out_ref,
                gwx_s, gwh_s, gb_s, uwx_s, uwh_s, ub_s):
    f, g = _F_IN, _F_OUT

    @pl.when(pl.program_id(0) == 0)
    def _expand_weights():
        gwx_s[...] = jnp.zeros_like(gwx_s)
        gwh_s[...] = jnp.zeros_like(gwh_s)
        uwx_s[...] = jnp.zeros_like(uwx_s)
        uwh_s[...] = jnp.zeros_like(uwh_s)

        def expand(pool_ref, n):
            w = e_ref[n, 0] * pool_ref[0]
            for d in range(1, _EMB):
                w = w + e_ref[n, d] * pool_ref[d]
            return w

        for p in range(_P):
            n0, n1 = 2 * p, 2 * p + 1
            wg0 = 0.5 * expand(gw_ref, n0)
            wg1 = 0.5 * expand(gw_ref, n1)
            bg0 = 0.5 * expand(gb_ref, n0)
            bg1 = 0.5 * expand(gb_ref, n1)
            wu0 = expand(uw_ref, n0)
            wu1 = expand(uw_ref, n1)
            bu0 = expand(ub_ref, n0)
            bu1 = expand(ub_ref, n1)

            gwx_s[p, 0 * g:1 * g, 0:f] = wg0[0:g, 0:f].astype(jnp.bfloat16)
            gwx_s[p, 1 * g:2 * g, f:2 * f] = wg1[0:g, 0:f].astype(jnp.bfloat16)
            gwx_s[p, 2 * g:3 * g, 0:f] = wg0[g:2 * g, 0:f].astype(jnp.bfloat16)
            gwx_s[p, 3 * g:4 * g, f:2 * f] = wg1[g:2 * g, 0:f].astype(jnp.bfloat16)
            gwh_s[p, 0 * g:1 * g, 0:g] = wg0[0:g, f:].astype(jnp.bfloat16)
            gwh_s[p, 1 * g:2 * g, g:2 * g] = wg1[0:g, f:].astype(jnp.bfloat16)
            gwh_s[p, 2 * g:3 * g, 0:g] = wg0[g:2 * g, f:].astype(jnp.bfloat16)
            gwh_s[p, 3 * g:4 * g, g:2 * g] = wg1[g:2 * g, f:].astype(jnp.bfloat16)
            gb_s[p, 0 * g:1 * g] = bg0[0:g]
            gb_s[p, 1 * g:2 * g] = bg1[0:g]
            gb_s[p, 2 * g:3 * g] = bg0[g:2 * g]
            gb_s[p, 3 * g:4 * g] = bg1[g:2 * g]

            uwx_s[p, 0:g, 0:f] = wu0[:, 0:f].astype(jnp.bfloat16)
            uwx_s[p, g:2 * g, f:2 * f] = wu1[:, 0:f].astype(jnp.bfloat16)
            uwh_s[p, 0:g, 0:g] = (0.5 * wu0[:, f:]).astype(jnp.bfloat16)
            uwh_s[p, g:2 * g, g:2 * g] = (0.5 * wu1[:, f:]).astype(jnp.bfloat16)
            ub_s[p, 0:g] = bu0
            ub_s[p, g:2 * g] = bu1

    xt = x_ref[...].astype(jnp.bfloat16)
    ht = h_ref[...]
    htb = ht.astype(jnp.bfloat16)

    vs = []
    for p in range(_P):
        xp = xt[2 * f * p:2 * f * (p + 1)]
        hp = ht[2 * g * p:2 * g * (p + 1)]
        hpb = htb[2 * g * p:2 * g * (p + 1)]

        ga = (jnp.dot(gwx_s[p], xp, preferred_element_type=jnp.float32)
              + jnp.dot(gwh_s[p], hpb, preferred_element_type=jnp.float32)
              + gb_s[p])
        t = jnp.tanh(ga)
        tz = t[:2 * g]
        tr = t[2 * g:]
        u = (tz * hp + hp).astype(jnp.bfloat16)
        hc = jnp.tanh(
            jnp.dot(uwx_s[p], xp, preferred_element_type=jnp.float32)
            + jnp.dot(uwh_s[p], u, preferred_element_type=jnp.float32)
            + ub_s[p])
        vs.append(jax.nn.relu((hc + hp) + tr * (hp - hc)))

    hall = jnp.concatenate(vs, axis=0)
    out_ref[...] = (
        jnp.dot(lwb_ref[...], hall, preferred_element_type=jnp.float32)
        + lb_ref[0, 0])


def kernel(x, e, h, gate_W, gate_b, update_W, update_b, lin_W, lin_b):
    B = x.shape[0]
    BB = 4096
    x2 = x.transpose(1, 2, 0).reshape(_N * _F_IN, B)
    h2 = h.transpose(1, 2, 0).reshape(_N * _F_OUT, B)
    gwT = gate_W.reshape(_EMB, _FC, 2 * _F_OUT).transpose(0, 2, 1)
    uwT = update_W.reshape(_EMB, _FC, _F_OUT).transpose(0, 2, 1)
    gbT = gate_b.reshape(_EMB, 2 * _F_OUT, 1)
    ubT = update_b.reshape(_EMB, _F_OUT, 1)
    lw_big = 0.5 * jnp.kron(jnp.eye(_N, dtype=x.dtype), lin_W)
    lb = lin_b.reshape(1, 1)

    grid = (B // BB,)
    full = lambda shape: pl.BlockSpec(shape, lambda i: (0,) * len(shape))
    out_t = pl.pallas_call(
        _agcrn_body,
        grid=grid,
        in_specs=[
            pl.BlockSpec((_N * _F_IN, BB), lambda i: (0, i)),
            pl.BlockSpec((_N * _F_OUT, BB), lambda i: (0, i)),
            full(e.shape),
            full(gwT.shape),
            full(gbT.shape),
            full(uwT.shape),
            full(ubT.shape),
            full(lw_big.shape),
            full(lb.shape),
        ],
        out_specs=pl.BlockSpec((_N, BB), lambda i: (0, i)),
        out_shape=jax.ShapeDtypeStruct((_N, B), x.dtype),
        scratch_shapes=[
            pltpu.VMEM((_P, 4 * _F_OUT, 2 * _F_IN), jnp.bfloat16),
            pltpu.VMEM((_P, 4 * _F_OUT, 2 * _F_OUT), jnp.bfloat16),
            pltpu.VMEM((_P, 4 * _F_OUT, 1), jnp.float32),
            pltpu.VMEM((_P, 2 * _F_OUT, 2 * _F_IN), jnp.bfloat16),
            pltpu.VMEM((_P, 2 * _F_OUT, 2 * _F_OUT), jnp.bfloat16),
            pltpu.VMEM((_P, 2 * _F_OUT, 1), jnp.float32),
        ],
        compiler_params=pltpu.CompilerParams(
            dimension_semantics=("arbitrary",)),
    )(x2, h2, e, gwT, gbT, uwT, ubT, lw_big, lb)
    return out_t.T.reshape(B, _N, 1)
